# baseline (speedup 1.0000x reference)
import functools

import jax
import jax.numpy as jnp
from jax import lax
from jax.experimental import pallas as pl
from jax.experimental.pallas import tpu as pltpu
from jax.experimental.pallas import tpu_sc as plsc

N = 10000
E = 320000
D = 128
H = 128
O = 64
G = 64

NC = 2
NS = 16
NW = NC * NS
EPW = E // NW
C = 80
NCHUNK = EPW // C
RPT = N // NS


def _sc_mp_body(compute_deg, *refs):
    if compute_deg:
        (x_hbm, src3_hbm, dst3_hbm, out_hbm, deg_hbm,
         acc, idx_s, idx_d, sem0, sem1, sem2, ssem0, ssem1, ssem2) = refs
    else:
        (x_hbm, src3_hbm, dst3_hbm, out_hbm,
         acc, idx_s, idx_d, sem0, sem1, sem2, ssem0, ssem1, ssem2) = refs
    gsems = (sem0, sem1, sem2)
    ssems = (ssem0, ssem1, ssem2)
    cid = lax.axis_index("c")
    sid = lax.axis_index("s")
    wid = sid * NC + cid
    base_row = sid * RPT
    zeros16 = jnp.zeros((16,), jnp.float32)

    pltpu.sync_copy(src3_hbm.at[wid], idx_s)
    pltpu.sync_copy(dst3_hbm.at[wid], idx_d)

    def g_start(t, rbuf, sem):
        pltpu.async_copy(x_hbm.at[idx_s.at[t]], rbuf, sem)

    def g_wait(t, rbuf, sem):
        pltpu.make_async_copy(x_hbm.at[idx_s.at[t]], rbuf, sem).wait()

    def s_add(t, rbuf):
        pltpu.sync_copy(rbuf, acc.at[idx_d.at[t]], add=True)

    def s_start(t, rbuf, sem):
        pltpu.async_copy(rbuf, acc.at[idx_d.at[t]], sem, add=True)

    def s_wait(t, rbuf, sem):
        pltpu.make_async_copy(rbuf, acc.at[idx_d.at[t]], sem).wait()

    def edge_phase(rows0, rows1, rows2):
        bufs = (rows0, rows1, rows2)

        def zb(i, _):
            for j in range(8):
                rows0[i, pl.ds(j * 16, 16)] = zeros16
            return 0

        lax.fori_loop(0, C, zb, 0)
        for k in range(RPT // C):
            pltpu.sync_copy(rows0, acc.at[pl.ds(base_row + k * C, C)])
        rem = RPT % C
        if rem:
            pltpu.sync_copy(rows0.at[pl.ds(0, rem)],
                            acc.at[pl.ds(base_row + (RPT // C) * C, rem)])
        plsc.subcore_barrier()

        g_start(0, bufs[0], gsems[0])
        g_start(1, bufs[1], gsems[1])

        def step3(i, _):
            for k in range(3):
                t = 3 * i + k
                b, gs, ss = bufs[k], gsems[k], ssems[k]
                kp = (k + 2) % 3
                g_wait(t, b, gs)
                s_start(t, b, ss)

                @pl.when(t + 2 < NCHUNK)
                def _():
                    @pl.when(t >= 1)
                    def _():
                        s_wait(t - 1, bufs[kp], ssems[kp])

                    g_start(t + 2, bufs[kp], gsems[kp])
            return 0

        lax.fori_loop(0, NCHUNK // 3, step3, 0)
        nm = (NCHUNK // 3) * 3
        assert NCHUNK - nm == 2
        s_wait(nm - 1, bufs[(nm - 1) % 3], ssems[(nm - 1) % 3])
        for t in range(nm, NCHUNK):
            b = t % 3
            g_wait(t, bufs[b], gsems[b])
            s_add(t, bufs[b])

    pl.run_scoped(edge_phase,
                  pltpu.VMEM((C, D), jnp.float32),
                  pltpu.VMEM((C, D), jnp.float32),
                  pltpu.VMEM((C, D), jnp.float32))

    plsc.subcore_barrier()

    acc_sl = acc.at[pl.ds(base_row, RPT)]
    out_sl = out_hbm.at[cid, pl.ds(base_row, RPT)]
    pltpu.async_copy(acc_sl, out_sl, sem0)

    if compute_deg:
        def deg_phase(deg_loc):
            def zd(i, _):
                deg_loc[pl.ds(i * 16, 16)] = zeros16
                return 0

            lax.fori_loop(0, N // 16, zd, 0)
            ones16 = jnp.ones((16,), jnp.float32)

            def hist(t, _):
                for j in range(C // 16):
                    dv = idx_d[t, pl.ds(j * 16, 16)]
                    plsc.addupdate_scatter(deg_loc, [dv], ones16)
                return 0

            lax.fori_loop(0, NCHUNK, hist, 0)
            pltpu.sync_copy(deg_loc, deg_hbm.at[cid, sid])

        pl.run_scoped(deg_phase, pltpu.VMEM((N,), jnp.float32))

    pltpu.make_async_copy(acc_sl, out_sl, sem0).wait()


@functools.lru_cache(maxsize=None)
def _mp_kernel(compute_deg):
    mesh = plsc.VectorSubcoreMesh(core_axis_name="c", subcore_axis_name="s")
    out_type = [jax.ShapeDtypeStruct((NC, N, D), jnp.float32)]
    scratch = [
        pltpu.VMEM_SHARED((N, D), jnp.float32),
        pltpu.VMEM((NCHUNK, C), jnp.int32),
        pltpu.VMEM((NCHUNK, C), jnp.int32),
    ]
    if compute_deg:
        out_type.append(jax.ShapeDtypeStruct((NC, NS, N), jnp.float32))
    scratch += [pltpu.SemaphoreType.DMA] * 6
    return pl.kernel(
        functools.partial(_sc_mp_body, compute_deg),
        out_type=tuple(out_type),
        mesh=mesh,
        scratch_types=scratch,
        compiler_params=pltpu.CompilerParams(use_tc_tiling_on_sc=False,
                                             needs_layout_passes=False),
    )


def _sc_mp(x, src, dst, compute_deg=False):
    src3 = src.reshape(NW, NCHUNK, C)
    dst3 = dst.reshape(NW, NCHUNK, C)
    if compute_deg:
        return _mp_kernel(True)(x, src3, dst3)
    return _mp_kernel(False)(x, src3, dst3)[0]


def _pre_body(x_ref, Wr_ref, b_ref, out_ref):
    out_ref[...] = (jnp.dot(x_ref[...], Wr_ref[...],
                            preferred_element_type=jnp.float32) + b_ref[...])


def _pre(x, Wr, b):
    return pl.pallas_call(
        _pre_body,
        out_shape=jax.ShapeDtypeStruct((N, H), jnp.float32),
    )(x, Wr, b)


def _dense_body(part_ref, deg_ref, xr_ref, Wl_ref, g_ref,
                be_ref, a_ref, out_ref):
    deg = jnp.sum(deg_ref[...], axis=(0, 1))
    inv = 1.0 / jnp.maximum(deg, 1.0)
    agg = (part_ref[0] + part_ref[1]) * inv[:, None]
    y = (jnp.dot(agg, Wl_ref[...], preferred_element_type=jnp.float32)
         + xr_ref[...])
    m = jnp.mean(y, axis=0)
    v = jnp.mean((y - m) ** 2, axis=0)
    z = (y - m) / jnp.sqrt(v + 1e-5) * g_ref[...] + be_ref[...]
    out_ref[...] = jnp.where(z > 0, z, a_ref[...] * z)


def _dense(part, deg, xr, Wl, g, be, a):
    return pl.pallas_call(
        _dense_body,
        out_shape=jax.ShapeDtypeStruct((N, H), jnp.float32),
    )(part, deg, xr, Wl, g, be, a)


def _final_body(part_ref, deg_ref, xr_ref, x2_ref, x1_ref, batch_ref,
                Wl_ref, g_ref, be_ref, a_ref,
                Wjk_ref, bjk_ref, Wf_ref, bf_ref, out_ref):
    deg = jnp.sum(deg_ref[...], axis=(0, 1))
    inv = 1.0 / jnp.maximum(deg, 1.0)
    agg = (part_ref[0] + part_ref[1]) * inv[:, None]
    y = (jnp.dot(agg, Wl_ref[...], preferred_element_type=jnp.float32)
         + xr_ref[...])
    m = jnp.mean(y, axis=0)
    v = jnp.mean((y - m) ** 2, axis=0)
    z = (y - m) / jnp.sqrt(v + 1e-5) * g_ref[...] + be_ref[...]
    x3 = jnp.where(z > 0, z, a_ref[...] * z)

    h = (jnp.dot(x1_ref[...], Wjk_ref[0], preferred_element_type=jnp.float32)
         + jnp.dot(x2_ref[...], Wjk_ref[1], preferred_element_type=jnp.float32)
         + jnp.dot(x3, Wjk_ref[2], preferred_element_type=jnp.float32)
         + bjk_ref[...])
    h = jnp.maximum(h, 0.0)

    gid = lax.broadcasted_iota(jnp.int32, (N, G), 1)
    onehot = (batch_ref[...][:, None] == gid).astype(jnp.float32)
    cnt = jnp.sum(onehot, axis=0)
    pooled = lax.dot_general(onehot, h, (((0,), (0,)), ((), ())),
                             preferred_element_type=jnp.float32)
    pooled = pooled / jnp.maximum(cnt, 1.0)[:, None]
    out_ref[...] = (jnp.dot(pooled, Wf_ref[...],
                            preferred_element_type=jnp.float32) + bf_ref[...])


def _final(part, deg, xr, x2, x1, batch, Wl, g, be, a, Wjk, bjk, Wf, bf):
    return pl.pallas_call(
        _final_body,
        out_shape=jax.ShapeDtypeStruct((G, O), jnp.float32),
    )(part, deg, xr, x2, x1, batch, Wl, g, be, a,
      Wjk.reshape(3, H, H), bjk, Wf, bf)


def kernel(x, edge_index, batch, Wl1, Wr1, b1, g1, be1, a1,
           Wl2, Wr2, b2, g2, be2, a2, Wl3, Wr3, b3, g3, be3, a3,
           Wjk, bjk, Wf, bf):
    src = edge_index[0]
    dst = edge_index[1]
    part1, deg = _sc_mp(x, src, dst, compute_deg=True)
    xr1 = _pre(x, Wr1, b1)
    x1 = _dense(part1, deg, xr1, Wl1, g1, be1, a1)
    part2 = _sc_mp(x1, src, dst)
    xr2 = _pre(x1, Wr2, b2)
    x2 = _dense(part2, deg, xr2, Wl2, g2, be2, a2)
    part3 = _sc_mp(x2, src, dst)
    xr3 = _pre(x2, Wr3, b3)
    return _final(part3, deg, xr3, x2, x1, batch,
                  Wl3, g3, be3, a3, Wjk, bjk, Wf, bf)


# speedup vs baseline: 1.0058x; 1.0058x over previous
import functools

import jax
import jax.numpy as jnp
from jax import lax
from jax.experimental import pallas as pl
from jax.experimental.pallas import tpu as pltpu
from jax.experimental.pallas import tpu_sc as plsc

N = 10000
E = 320000
D = 128
H = 128
O = 64
G = 64

NC = 2
NS = 16
NW = NC * NS
EPW = E // NW
C = 80
NCHUNK = EPW // C
RPT = N // NS


def _sc_mp_body(compute_deg, *refs):
    if compute_deg:
        (x_hbm, src3_hbm, dst3_hbm, out_hbm, deg_hbm,
         acc, idx_s, idx_d, sem0, sem1, sem2, ssem0, ssem1, ssem2) = refs
    else:
        (x_hbm, src3_hbm, dst3_hbm, out_hbm,
         acc, idx_s, idx_d, sem0, sem1, sem2, ssem0, ssem1, ssem2) = refs
    gsems = (sem0, sem1, sem2)
    ssems = (ssem0, ssem1, ssem2)
    cid = lax.axis_index("c")
    sid = lax.axis_index("s")
    wid = sid * NC + cid
    base_row = sid * RPT
    zeros16 = jnp.zeros((16,), jnp.float32)

    pltpu.sync_copy(src3_hbm.at[wid], idx_s)
    pltpu.sync_copy(dst3_hbm.at[wid], idx_d)

    def g_start(t, rbuf, sem):
        pltpu.async_copy(x_hbm.at[idx_s.at[t]], rbuf, sem)

    def g_wait(t, rbuf, sem):
        pltpu.make_async_copy(x_hbm.at[idx_s.at[t]], rbuf, sem).wait()

    def s_add(t, rbuf):
        pltpu.sync_copy(rbuf, acc.at[idx_d.at[t]], add=True)

    def s_start(t, rbuf, sem):
        pltpu.async_copy(rbuf, acc.at[idx_d.at[t]], sem, add=True)

    def s_wait(t, rbuf, sem):
        pltpu.make_async_copy(rbuf, acc.at[idx_d.at[t]], sem).wait()

    def edge_phase(rows0, rows1, rows2):
        bufs = (rows0, rows1, rows2)

        def zb(i, _):
            for j in range(8):
                rows0[i, pl.ds(j * 16, 16)] = zeros16
            return 0

        lax.fori_loop(0, C, zb, 0)
        for k in range(RPT // C):
            pltpu.sync_copy(rows0, acc.at[pl.ds(base_row + k * C, C)])
        rem = RPT % C
        if rem:
            pltpu.sync_copy(rows0.at[pl.ds(0, rem)],
                            acc.at[pl.ds(base_row + (RPT // C) * C, rem)])
        plsc.subcore_barrier()

        g_start(0, bufs[0], gsems[0])
        g_start(1, bufs[1], gsems[1])

        def step3(i, _):
            for k in range(3):
                t = 3 * i + k
                b, gs, ss = bufs[k], gsems[k], ssems[k]
                kp = (k + 2) % 3
                g_wait(t, b, gs)
                s_start(t, b, ss)

                @pl.when(t + 2 < NCHUNK)
                def _():
                    @pl.when(t >= 1)
                    def _():
                        s_wait(t - 1, bufs[kp], ssems[kp])

                    g_start(t + 2, bufs[kp], gsems[kp])
            return 0

        lax.fori_loop(0, NCHUNK // 3, step3, 0)
        nm = (NCHUNK // 3) * 3
        assert NCHUNK - nm == 2
        s_wait(nm - 1, bufs[(nm - 1) % 3], ssems[(nm - 1) % 3])
        for t in range(nm, NCHUNK):
            b = t % 3
            g_wait(t, bufs[b], gsems[b])
            s_add(t, bufs[b])

    pl.run_scoped(edge_phase,
                  pltpu.VMEM((C, D), jnp.float32),
                  pltpu.VMEM((C, D), jnp.float32),
                  pltpu.VMEM((C, D), jnp.float32))

    plsc.subcore_barrier()

    acc_sl = acc.at[pl.ds(base_row, RPT)]
    out_sl = out_hbm.at[cid, pl.ds(base_row, RPT)]
    pltpu.async_copy(acc_sl, out_sl, sem0)

    if compute_deg:
        def deg_phase(deg_loc):
            def zd(i, _):
                deg_loc[pl.ds(i * 16, 16)] = zeros16
                return 0

            lax.fori_loop(0, N // 16, zd, 0)
            ones16 = jnp.ones((16,), jnp.float32)

            def hist(t, _):
                for j in range(C // 16):
                    dv = idx_d[t, pl.ds(j * 16, 16)]
                    plsc.addupdate_scatter(deg_loc, [dv], ones16)
                return 0

            lax.fori_loop(0, NCHUNK, hist, 0)
            pltpu.sync_copy(deg_loc, deg_hbm.at[cid, sid])

        pl.run_scoped(deg_phase, pltpu.VMEM((N,), jnp.float32))

    pltpu.make_async_copy(acc_sl, out_sl, sem0).wait()


@functools.lru_cache(maxsize=None)
def _mp_kernel(compute_deg):
    mesh = plsc.VectorSubcoreMesh(core_axis_name="c", subcore_axis_name="s")
    out_type = [jax.ShapeDtypeStruct((NC, N, D), jnp.float32)]
    scratch = [
        pltpu.VMEM_SHARED((N, D), jnp.float32),
        pltpu.VMEM((NCHUNK, C), jnp.int32),
        pltpu.VMEM((NCHUNK, C), jnp.int32),
    ]
    if compute_deg:
        out_type.append(jax.ShapeDtypeStruct((NC, NS, N), jnp.float32))
    scratch += [pltpu.SemaphoreType.DMA] * 6
    return pl.kernel(
        functools.partial(_sc_mp_body, compute_deg),
        out_type=tuple(out_type),
        mesh=mesh,
        scratch_types=scratch,
        compiler_params=pltpu.CompilerParams(use_tc_tiling_on_sc=False,
                                             needs_layout_passes=False),
    )


def _sc_mp(x, src, dst, compute_deg=False):
    src3 = src.reshape(NW, NCHUNK, C)
    dst3 = dst.reshape(NW, NCHUNK, C)
    if compute_deg:
        return _mp_kernel(True)(x, src3, dst3)
    return _mp_kernel(False)(x, src3, dst3)[0]


def _dense_body(part_ref, deg_ref, x_ref, Wl_ref, Wr_ref, b_ref, g_ref,
                be_ref, a_ref, out_ref):
    deg = jnp.sum(deg_ref[...], axis=(0, 1))
    inv = 1.0 / jnp.maximum(deg, 1.0)
    agg = (part_ref[0] + part_ref[1]) * inv[:, None]
    y = (jnp.dot(agg, Wl_ref[...], preferred_element_type=jnp.float32)
         + jnp.dot(x_ref[...], Wr_ref[...], preferred_element_type=jnp.float32)
         + b_ref[...])
    m = jnp.mean(y, axis=0)
    v = jnp.mean((y - m) ** 2, axis=0)
    z = (y - m) / jnp.sqrt(v + 1e-5) * g_ref[...] + be_ref[...]
    out_ref[...] = jnp.where(z > 0, z, a_ref[...] * z)


def _dense(part, deg, x, Wl, Wr, b, g, be, a):
    return pl.pallas_call(
        _dense_body,
        out_shape=jax.ShapeDtypeStruct((N, H), jnp.float32),
    )(part, deg, x, Wl, Wr, b, g, be, a)


def _final_body(part_ref, deg_ref, x2_ref, x1_ref, batch_ref,
                Wl_ref, Wr_ref, b_ref, g_ref, be_ref, a_ref,
                Wjk_ref, bjk_ref, Wf_ref, bf_ref, out_ref):
    deg = jnp.sum(deg_ref[...], axis=(0, 1))
    inv = 1.0 / jnp.maximum(deg, 1.0)
    agg = (part_ref[0] + part_ref[1]) * inv[:, None]
    y = (jnp.dot(agg, Wl_ref[...], preferred_element_type=jnp.float32)
         + jnp.dot(x2_ref[...], Wr_ref[...], preferred_element_type=jnp.float32)
         + b_ref[...])
    m = jnp.mean(y, axis=0)
    v = jnp.mean((y - m) ** 2, axis=0)
    z = (y - m) / jnp.sqrt(v + 1e-5) * g_ref[...] + be_ref[...]
    x3 = jnp.where(z > 0, z, a_ref[...] * z)

    h = (jnp.dot(x1_ref[...], Wjk_ref[0], preferred_element_type=jnp.float32)
         + jnp.dot(x2_ref[...], Wjk_ref[1], preferred_element_type=jnp.float32)
         + jnp.dot(x3, Wjk_ref[2], preferred_element_type=jnp.float32)
         + bjk_ref[...])
    h = jnp.maximum(h, 0.0)

    gid = lax.broadcasted_iota(jnp.int32, (N, G), 1)
    onehot = (batch_ref[...][:, None] == gid).astype(jnp.float32)
    cnt = jnp.sum(onehot, axis=0)
    pooled = lax.dot_general(onehot, h, (((0,), (0,)), ((), ())),
                             preferred_element_type=jnp.float32)
    pooled = pooled / jnp.maximum(cnt, 1.0)[:, None]
    out_ref[...] = (jnp.dot(pooled, Wf_ref[...],
                            preferred_element_type=jnp.float32) + bf_ref[...])


def _final(part, deg, x2, x1, batch, Wl, Wr, b, g, be, a, Wjk, bjk, Wf, bf):
    return pl.pallas_call(
        _final_body,
        out_shape=jax.ShapeDtypeStruct((G, O), jnp.float32),
    )(part, deg, x2, x1, batch, Wl, Wr, b, g, be, a,
      Wjk.reshape(3, H, H), bjk, Wf, bf)


def kernel(x, edge_index, batch, Wl1, Wr1, b1, g1, be1, a1,
           Wl2, Wr2, b2, g2, be2, a2, Wl3, Wr3, b3, g3, be3, a3,
           Wjk, bjk, Wf, bf):
    src = edge_index[0]
    dst = edge_index[1]
    part1, deg = _sc_mp(x, src, dst, compute_deg=True)
    x1 = _dense(part1, deg, x, Wl1, Wr1, b1, g1, be1, a1)
    part2 = _sc_mp(x1, src, dst)
    x2 = _dense(part2, deg, x1, Wl2, Wr2, b2, g2, be2, a2)
    part3 = _sc_mp(x2, src, dst)
    return _final(part3, deg, x2, x1, batch,
                  Wl3, Wr3, b3, g3, be3, a3, Wjk, bjk, Wf, bf)


# speedup vs baseline: 1.0069x; 1.0011x over previous
import functools

import jax
import jax.numpy as jnp
from jax import lax
from jax.experimental import pallas as pl
from jax.experimental.pallas import tpu as pltpu
from jax.experimental.pallas import tpu_sc as plsc

N = 10000
E = 320000
D = 128
H = 128
O = 64
G = 64

NC = 2
NS = 16
NW = NC * NS
EPW = E // NW
C = 80
NCHUNK = EPW // C
RPT = N // NS


def _sc_mp_body(compute_deg, *refs):
    if compute_deg:
        (x_hbm, src3_hbm, dst3_hbm, out_hbm, deg_hbm,
         acc, idx_s, idx_d, sem0, sem1, sem2, ssem0, ssem1, ssem2) = refs
    else:
        (x_hbm, src3_hbm, dst3_hbm, out_hbm,
         acc, idx_s, idx_d, sem0, sem1, sem2, ssem0, ssem1, ssem2) = refs
    gsems = (sem0, sem1, sem2)
    ssems = (ssem0, ssem1, ssem2)
    cid = lax.axis_index("c")
    sid = lax.axis_index("s")
    wid = sid * NC + cid
    base_row = sid * RPT
    zeros16 = jnp.zeros((16,), jnp.float32)

    pltpu.sync_copy(src3_hbm.at[wid], idx_s)
    pltpu.sync_copy(dst3_hbm.at[wid], idx_d)

    CH = C // 2

    def g_start(t, rbuf, sem):
        pltpu.async_copy(x_hbm.at[idx_s.at[t, pl.ds(0, CH)]],
                         rbuf.at[pl.ds(0, CH)], sem)
        pltpu.async_copy(x_hbm.at[idx_s.at[t, pl.ds(CH, CH)]],
                         rbuf.at[pl.ds(CH, CH)], sem)

    def g_wait(t, rbuf, sem):
        pltpu.make_async_copy(x_hbm.at[idx_s.at[t, pl.ds(0, CH)]],
                              rbuf.at[pl.ds(0, CH)], sem).wait()
        pltpu.make_async_copy(x_hbm.at[idx_s.at[t, pl.ds(CH, CH)]],
                              rbuf.at[pl.ds(CH, CH)], sem).wait()

    def s_add(t, rbuf):
        pltpu.sync_copy(rbuf, acc.at[idx_d.at[t]], add=True)

    def s_start(t, rbuf, sem):
        pltpu.async_copy(rbuf, acc.at[idx_d.at[t]], sem, add=True)

    def s_wait(t, rbuf, sem):
        pltpu.make_async_copy(rbuf, acc.at[idx_d.at[t]], sem).wait()

    def edge_phase(rows0, rows1, rows2):
        bufs = (rows0, rows1, rows2)

        def zb(i, _):
            for j in range(8):
                rows0[i, pl.ds(j * 16, 16)] = zeros16
            return 0

        lax.fori_loop(0, C, zb, 0)
        for k in range(RPT // C):
            pltpu.sync_copy(rows0, acc.at[pl.ds(base_row + k * C, C)])
        rem = RPT % C
        if rem:
            pltpu.sync_copy(rows0.at[pl.ds(0, rem)],
                            acc.at[pl.ds(base_row + (RPT // C) * C, rem)])
        plsc.subcore_barrier()

        g_start(0, bufs[0], gsems[0])
        g_start(1, bufs[1], gsems[1])

        def step3(i, _):
            for k in range(3):
                t = 3 * i + k
                b, gs, ss = bufs[k], gsems[k], ssems[k]
                kp = (k + 2) % 3
                g_wait(t, b, gs)
                s_start(t, b, ss)

                @pl.when(t + 2 < NCHUNK)
                def _():
                    @pl.when(t >= 1)
                    def _():
                        s_wait(t - 1, bufs[kp], ssems[kp])

                    g_start(t + 2, bufs[kp], gsems[kp])
            return 0

        lax.fori_loop(0, NCHUNK // 3, step3, 0)
        nm = (NCHUNK // 3) * 3
        assert NCHUNK - nm == 2
        s_wait(nm - 1, bufs[(nm - 1) % 3], ssems[(nm - 1) % 3])
        for t in range(nm, NCHUNK):
            b = t % 3
            g_wait(t, bufs[b], gsems[b])
            s_add(t, bufs[b])

    pl.run_scoped(edge_phase,
                  pltpu.VMEM((C, D), jnp.float32),
                  pltpu.VMEM((C, D), jnp.float32),
                  pltpu.VMEM((C, D), jnp.float32))

    plsc.subcore_barrier()

    acc_sl = acc.at[pl.ds(base_row, RPT)]
    out_sl = out_hbm.at[cid, pl.ds(base_row, RPT)]
    pltpu.async_copy(acc_sl, out_sl, sem0)

    if compute_deg:
        def deg_phase(deg_loc):
            def zd(i, _):
                deg_loc[pl.ds(i * 16, 16)] = zeros16
                return 0

            lax.fori_loop(0, N // 16, zd, 0)
            ones16 = jnp.ones((16,), jnp.float32)

            def hist(t, _):
                for j in range(C // 16):
                    dv = idx_d[t, pl.ds(j * 16, 16)]
                    plsc.addupdate_scatter(deg_loc, [dv], ones16)
                return 0

            lax.fori_loop(0, NCHUNK, hist, 0)
            pltpu.sync_copy(deg_loc, deg_hbm.at[cid, sid])

        pl.run_scoped(deg_phase, pltpu.VMEM((N,), jnp.float32))

    pltpu.make_async_copy(acc_sl, out_sl, sem0).wait()


@functools.lru_cache(maxsize=None)
def _mp_kernel(compute_deg):
    mesh = plsc.VectorSubcoreMesh(core_axis_name="c", subcore_axis_name="s")
    out_type = [jax.ShapeDtypeStruct((NC, N, D), jnp.float32)]
    scratch = [
        pltpu.VMEM_SHARED((N, D), jnp.float32),
        pltpu.VMEM((NCHUNK, C), jnp.int32),
        pltpu.VMEM((NCHUNK, C), jnp.int32),
    ]
    if compute_deg:
        out_type.append(jax.ShapeDtypeStruct((NC, NS, N), jnp.float32))
    scratch += [pltpu.SemaphoreType.DMA] * 6
    return pl.kernel(
        functools.partial(_sc_mp_body, compute_deg),
        out_type=tuple(out_type),
        mesh=mesh,
        scratch_types=scratch,
        compiler_params=pltpu.CompilerParams(use_tc_tiling_on_sc=False,
                                             needs_layout_passes=False),
    )


def _sc_mp(x, src, dst, compute_deg=False):
    src3 = src.reshape(NW, NCHUNK, C)
    dst3 = dst.reshape(NW, NCHUNK, C)
    if compute_deg:
        return _mp_kernel(True)(x, src3, dst3)
    return _mp_kernel(False)(x, src3, dst3)[0]


def _dense_body(part_ref, deg_ref, x_ref, Wl_ref, Wr_ref, b_ref, g_ref,
                be_ref, a_ref, out_ref):
    deg = jnp.sum(deg_ref[...], axis=(0, 1))
    inv = 1.0 / jnp.maximum(deg, 1.0)
    agg = (part_ref[0] + part_ref[1]) * inv[:, None]
    y = (jnp.dot(agg, Wl_ref[...], preferred_element_type=jnp.float32)
         + jnp.dot(x_ref[...], Wr_ref[...], preferred_element_type=jnp.float32)
         + b_ref[...])
    m = jnp.mean(y, axis=0)
    v = jnp.mean((y - m) ** 2, axis=0)
    z = (y - m) / jnp.sqrt(v + 1e-5) * g_ref[...] + be_ref[...]
    out_ref[...] = jnp.where(z > 0, z, a_ref[...] * z)


def _dense(part, deg, x, Wl, Wr, b, g, be, a):
    return pl.pallas_call(
        _dense_body,
        out_shape=jax.ShapeDtypeStruct((N, H), jnp.float32),
    )(part, deg, x, Wl, Wr, b, g, be, a)


def _final_body(part_ref, deg_ref, x2_ref, x1_ref, batch_ref,
                Wl_ref, Wr_ref, b_ref, g_ref, be_ref, a_ref,
                Wjk_ref, bjk_ref, Wf_ref, bf_ref, out_ref):
    deg = jnp.sum(deg_ref[...], axis=(0, 1))
    inv = 1.0 / jnp.maximum(deg, 1.0)
    agg = (part_ref[0] + part_ref[1]) * inv[:, None]
    y = (jnp.dot(agg, Wl_ref[...], preferred_element_type=jnp.float32)
         + jnp.dot(x2_ref[...], Wr_ref[...], preferred_element_type=jnp.float32)
         + b_ref[...])
    m = jnp.mean(y, axis=0)
    v = jnp.mean((y - m) ** 2, axis=0)
    z = (y - m) / jnp.sqrt(v + 1e-5) * g_ref[...] + be_ref[...]
    x3 = jnp.where(z > 0, z, a_ref[...] * z)

    h = (jnp.dot(x1_ref[...], Wjk_ref[0], preferred_element_type=jnp.float32)
         + jnp.dot(x2_ref[...], Wjk_ref[1], preferred_element_type=jnp.float32)
         + jnp.dot(x3, Wjk_ref[2], preferred_element_type=jnp.float32)
         + bjk_ref[...])
    h = jnp.maximum(h, 0.0)

    gid = lax.broadcasted_iota(jnp.int32, (N, G), 1)
    onehot = (batch_ref[...][:, None] == gid).astype(jnp.float32)
    cnt = jnp.sum(onehot, axis=0)
    pooled = lax.dot_general(onehot, h, (((0,), (0,)), ((), ())),
                             preferred_element_type=jnp.float32)
    pooled = pooled / jnp.maximum(cnt, 1.0)[:, None]
    out_ref[...] = (jnp.dot(pooled, Wf_ref[...],
                            preferred_element_type=jnp.float32) + bf_ref[...])


def _final(part, deg, x2, x1, batch, Wl, Wr, b, g, be, a, Wjk, bjk, Wf, bf):
    return pl.pallas_call(
        _final_body,
        out_shape=jax.ShapeDtypeStruct((G, O), jnp.float32),
    )(part, deg, x2, x1, batch, Wl, Wr, b, g, be, a,
      Wjk.reshape(3, H, H), bjk, Wf, bf)


def kernel(x, edge_index, batch, Wl1, Wr1, b1, g1, be1, a1,
           Wl2, Wr2, b2, g2, be2, a2, Wl3, Wr3, b3, g3, be3, a3,
           Wjk, bjk, Wf, bf):
    src = edge_index[0]
    dst = edge_index[1]
    part1, deg = _sc_mp(x, src, dst, compute_deg=True)
    x1 = _dense(part1, deg, x, Wl1, Wr1, b1, g1, be1, a1)
    part2 = _sc_mp(x1, src, dst)
    x2 = _dense(part2, deg, x1, Wl2, Wr2, b2, g2, be2, a2)
    part3 = _sc_mp(x2, src, dst)
    return _final(part3, deg, x2, x1, batch,
                  Wl3, Wr3, b3, g3, be3, a3, Wjk, bjk, Wf, bf)


# speedup vs baseline: 1.0268x; 1.0198x over previous
import functools

import jax
import jax.numpy as jnp
from jax import lax
from jax.experimental import pallas as pl
from jax.experimental.pallas import tpu as pltpu
from jax.experimental.pallas import tpu_sc as plsc

N = 10000
E = 320000
D = 128
H = 128
O = 64
G = 64

NC = 2
NS = 16
NW = NC * NS
EPW = E // NW
C = 80
NCHUNK = EPW // C
RPT = N // NS


def _sc_mp_body(compute_deg, *refs):
    if compute_deg:
        (x_hbm, src3_hbm, dst3_hbm, out_hbm, deg_hbm,
         acc, idx_s, idx_d, sem0, sem1, sem2, ssem0, ssem1, ssem2) = refs
    else:
        (x_hbm, src3_hbm, dst3_hbm, out_hbm,
         acc, idx_s, idx_d, sem0, sem1, sem2, ssem0, ssem1, ssem2) = refs
    gsems = (sem0, sem1, sem2)
    ssems = (ssem0, ssem1, ssem2)
    cid = lax.axis_index("c")
    sid = lax.axis_index("s")
    wid = sid * NC + cid
    base_row = sid * RPT
    zeros16 = jnp.zeros((16,), jnp.float32)

    pltpu.async_copy(src3_hbm.at[wid], idx_s, sem0)
    pltpu.async_copy(dst3_hbm.at[wid], idx_d, sem1)

    CH = C // 2

    def g_start(t, rbuf, sem):
        pltpu.async_copy(x_hbm.at[idx_s.at[t, pl.ds(0, CH)]],
                         rbuf.at[pl.ds(0, CH)], sem)
        pltpu.async_copy(x_hbm.at[idx_s.at[t, pl.ds(CH, CH)]],
                         rbuf.at[pl.ds(CH, CH)], sem)

    def g_wait(t, rbuf, sem):
        pltpu.make_async_copy(x_hbm.at[idx_s.at[t, pl.ds(0, CH)]],
                              rbuf.at[pl.ds(0, CH)], sem).wait()
        pltpu.make_async_copy(x_hbm.at[idx_s.at[t, pl.ds(CH, CH)]],
                              rbuf.at[pl.ds(CH, CH)], sem).wait()

    def s_add(t, rbuf):
        pltpu.sync_copy(rbuf, acc.at[idx_d.at[t]], add=True)

    def s_start(t, rbuf, sem):
        pltpu.async_copy(rbuf, acc.at[idx_d.at[t]], sem, add=True)

    def s_wait(t, rbuf, sem):
        pltpu.make_async_copy(rbuf, acc.at[idx_d.at[t]], sem).wait()

    def edge_phase(rows0, rows1, rows2):
        bufs = (rows0, rows1, rows2)

        def zb(i, _):
            for j in range(8):
                rows0[i, pl.ds(j * 16, 16)] = zeros16
            return 0

        lax.fori_loop(0, C, zb, 0)
        for k in range(RPT // C):
            pltpu.async_copy(rows0, acc.at[pl.ds(base_row + k * C, C)], sem2)
        rem = RPT % C
        if rem:
            pltpu.async_copy(rows0.at[pl.ds(0, rem)],
                             acc.at[pl.ds(base_row + (RPT // C) * C, rem)],
                             sem2)
        for k in range(RPT // C):
            pltpu.make_async_copy(rows0, acc.at[pl.ds(base_row + k * C, C)],
                                  sem2).wait()
        if rem:
            pltpu.make_async_copy(rows0.at[pl.ds(0, rem)],
                                  acc.at[pl.ds(base_row + (RPT // C) * C, rem)],
                                  sem2).wait()
        pltpu.make_async_copy(src3_hbm.at[wid], idx_s, sem0).wait()
        pltpu.make_async_copy(dst3_hbm.at[wid], idx_d, sem1).wait()
        plsc.subcore_barrier()

        g_start(0, bufs[0], gsems[0])
        g_start(1, bufs[1], gsems[1])

        def step3(i, _):
            for k in range(3):
                t = 3 * i + k
                b, gs, ss = bufs[k], gsems[k], ssems[k]
                kp = (k + 2) % 3
                g_wait(t, b, gs)
                s_start(t, b, ss)

                @pl.when(t + 2 < NCHUNK)
                def _():
                    @pl.when(t >= 1)
                    def _():
                        s_wait(t - 1, bufs[kp], ssems[kp])

                    g_start(t + 2, bufs[kp], gsems[kp])
            return 0

        lax.fori_loop(0, NCHUNK // 3, step3, 0)
        nm = (NCHUNK // 3) * 3
        assert NCHUNK - nm == 2
        s_wait(nm - 1, bufs[(nm - 1) % 3], ssems[(nm - 1) % 3])
        for t in range(nm, NCHUNK):
            b = t % 3
            g_wait(t, bufs[b], gsems[b])
            s_add(t, bufs[b])

    pl.run_scoped(edge_phase,
                  pltpu.VMEM((C, D), jnp.float32),
                  pltpu.VMEM((C, D), jnp.float32),
                  pltpu.VMEM((C, D), jnp.float32))

    plsc.subcore_barrier()

    acc_sl = acc.at[pl.ds(base_row, RPT)]
    out_sl = out_hbm.at[cid, pl.ds(base_row, RPT)]
    pltpu.async_copy(acc_sl, out_sl, sem0)

    if compute_deg:
        def deg_phase(deg_loc):
            def zd(i, _):
                deg_loc[pl.ds(i * 16, 16)] = zeros16
                return 0

            lax.fori_loop(0, N // 16, zd, 0)
            ones16 = jnp.ones((16,), jnp.float32)

            def hist(t, _):
                for j in range(C // 16):
                    dv = idx_d[t, pl.ds(j * 16, 16)]
                    plsc.addupdate_scatter(deg_loc, [dv], ones16)
                return 0

            lax.fori_loop(0, NCHUNK, hist, 0)
            pltpu.sync_copy(deg_loc, deg_hbm.at[cid, sid])

        pl.run_scoped(deg_phase, pltpu.VMEM((N,), jnp.float32))

    pltpu.make_async_copy(acc_sl, out_sl, sem0).wait()


@functools.lru_cache(maxsize=None)
def _mp_kernel(compute_deg):
    mesh = plsc.VectorSubcoreMesh(core_axis_name="c", subcore_axis_name="s")
    out_type = [jax.ShapeDtypeStruct((NC, N, D), jnp.float32)]
    scratch = [
        pltpu.VMEM_SHARED((N, D), jnp.float32),
        pltpu.VMEM((NCHUNK, C), jnp.int32),
        pltpu.VMEM((NCHUNK, C), jnp.int32),
    ]
    if compute_deg:
        out_type.append(jax.ShapeDtypeStruct((NC, NS, N), jnp.float32))
    scratch += [pltpu.SemaphoreType.DMA] * 6
    return pl.kernel(
        functools.partial(_sc_mp_body, compute_deg),
        out_type=tuple(out_type),
        mesh=mesh,
        scratch_types=scratch,
        compiler_params=pltpu.CompilerParams(use_tc_tiling_on_sc=False,
                                             needs_layout_passes=False),
    )


def _sc_mp(x, src, dst, compute_deg=False):
    src3 = src.reshape(NW, NCHUNK, C)
    dst3 = dst.reshape(NW, NCHUNK, C)
    if compute_deg:
        return _mp_kernel(True)(x, src3, dst3)
    return _mp_kernel(False)(x, src3, dst3)[0]


def _dense_body(part_ref, deg_ref, x_ref, Wl_ref, Wr_ref, b_ref, g_ref,
                be_ref, a_ref, out_ref):
    deg = jnp.sum(deg_ref[...], axis=(0, 1))
    inv = 1.0 / jnp.maximum(deg, 1.0)
    agg = (part_ref[0] + part_ref[1]) * inv[:, None]
    y = (jnp.dot(agg, Wl_ref[...], preferred_element_type=jnp.float32)
         + jnp.dot(x_ref[...], Wr_ref[...], preferred_element_type=jnp.float32)
         + b_ref[...])
    m = jnp.mean(y, axis=0)
    v = jnp.mean((y - m) ** 2, axis=0)
    z = (y - m) / jnp.sqrt(v + 1e-5) * g_ref[...] + be_ref[...]
    out_ref[...] = jnp.where(z > 0, z, a_ref[...] * z)


def _dense(part, deg, x, Wl, Wr, b, g, be, a):
    return pl.pallas_call(
        _dense_body,
        out_shape=jax.ShapeDtypeStruct((N, H), jnp.float32),
    )(part, deg, x, Wl, Wr, b, g, be, a)


def _final_body(part_ref, deg_ref, x2_ref, x1_ref, batch_ref,
                Wl_ref, Wr_ref, b_ref, g_ref, be_ref, a_ref,
                Wjk_ref, bjk_ref, Wf_ref, bf_ref, out_ref):
    deg = jnp.sum(deg_ref[...], axis=(0, 1))
    inv = 1.0 / jnp.maximum(deg, 1.0)
    agg = (part_ref[0] + part_ref[1]) * inv[:, None]
    y = (jnp.dot(agg, Wl_ref[...], preferred_element_type=jnp.float32)
         + jnp.dot(x2_ref[...], Wr_ref[...], preferred_element_type=jnp.float32)
         + b_ref[...])
    m = jnp.mean(y, axis=0)
    v = jnp.mean((y - m) ** 2, axis=0)
    z = (y - m) / jnp.sqrt(v + 1e-5) * g_ref[...] + be_ref[...]
    x3 = jnp.where(z > 0, z, a_ref[...] * z)

    h = (jnp.dot(x1_ref[...], Wjk_ref[0], preferred_element_type=jnp.float32)
         + jnp.dot(x2_ref[...], Wjk_ref[1], preferred_element_type=jnp.float32)
         + jnp.dot(x3, Wjk_ref[2], preferred_element_type=jnp.float32)
         + bjk_ref[...])
    h = jnp.maximum(h, 0.0)

    gid = lax.broadcasted_iota(jnp.int32, (N, G), 1)
    onehot = (batch_ref[...][:, None] == gid).astype(jnp.float32)
    cnt = jnp.sum(onehot, axis=0)
    pooled = lax.dot_general(onehot, h, (((0,), (0,)), ((), ())),
                             preferred_element_type=jnp.float32)
    pooled = pooled / jnp.maximum(cnt, 1.0)[:, None]
    out_ref[...] = (jnp.dot(pooled, Wf_ref[...],
                            preferred_element_type=jnp.float32) + bf_ref[...])


def _final(part, deg, x2, x1, batch, Wl, Wr, b, g, be, a, Wjk, bjk, Wf, bf):
    return pl.pallas_call(
        _final_body,
        out_shape=jax.ShapeDtypeStruct((G, O), jnp.float32),
    )(part, deg, x2, x1, batch, Wl, Wr, b, g, be, a,
      Wjk.reshape(3, H, H), bjk, Wf, bf)


def kernel(x, edge_index, batch, Wl1, Wr1, b1, g1, be1, a1,
           Wl2, Wr2, b2, g2, be2, a2, Wl3, Wr3, b3, g3, be3, a3,
           Wjk, bjk, Wf, bf):
    src = edge_index[0]
    dst = edge_index[1]
    part1, deg = _sc_mp(x, src, dst, compute_deg=True)
    x1 = _dense(part1, deg, x, Wl1, Wr1, b1, g1, be1, a1)
    part2 = _sc_mp(x1, src, dst)
    x2 = _dense(part2, deg, x1, Wl2, Wr2, b2, g2, be2, a2)
    part3 = _sc_mp(x2, src, dst)
    return _final(part3, deg, x2, x1, batch,
                  Wl3, Wr3, b3, g3, be3, a3, Wjk, bjk, Wf, bf)
